# baseline (speedup 1.0000x reference)
import functools

import jax
import jax.numpy as jnp
from jax import lax
from jax.experimental import pallas as pl
from jax.experimental.pallas import tpu as pltpu
from jax.experimental.pallas import tpu_sc as plsc

H = 1024
MOE_I = 512
E = 64
N_GROUP = 8
GROUP_SIZE = E // N_GROUP
TOPK_GROUP = 4
SCALE = 2.5

R = 256
RT = 256

_SC_NC = 2
_SC_NS = 16
_SC_NW = _SC_NC * _SC_NS


def _sc_worker_base(bpw):
    wid = lax.axis_index("s") * _SC_NC + lax.axis_index("c")
    return wid * bpw


def _sc_gather_body(bpw, table_hbm, idx_hbm, out_hbm, idx_v, rows_v, sem):
    base = _sc_worker_base(bpw)
    pltpu.sync_copy(idx_hbm.at[pl.ds(base, bpw)], idx_v)
    pltpu.async_copy(table_hbm.at[idx_v], rows_v, sem).wait()
    pltpu.sync_copy(rows_v, out_hbm.at[pl.ds(base, bpw)])


def _sc_scatter_body(bpw, rows_hbm, idx_hbm, out_hbm, idx_v, rows_v, sem):
    base = _sc_worker_base(bpw)
    pltpu.sync_copy(idx_hbm.at[pl.ds(base, bpw)], idx_v)
    pltpu.sync_copy(rows_hbm.at[pl.ds(base, bpw)], rows_v)
    pltpu.async_copy(rows_v, out_hbm.at[idx_v], sem).wait()


def _sc_permute(table, idx, scatter):
    n, d = table.shape
    bpw = n // _SC_NW
    body = functools.partial(_sc_scatter_body if scatter else _sc_gather_body,
                             bpw)
    mesh = plsc.VectorSubcoreMesh(core_axis_name="c", subcore_axis_name="s")
    return pl.kernel(
        body,
        mesh=mesh,
        out_type=jax.ShapeDtypeStruct((n, d), table.dtype),
        scratch_types=[
            pltpu.VMEM((bpw,), jnp.int32),
            pltpu.VMEM((bpw, d), table.dtype),
            pltpu.SemaphoreType.DMA,
        ],
    )(table, idx)


def _router_body(hs_ref, rw_ref, eid_ref, w_ref):
    x = hs_ref[...]
    rw = rw_ref[...]
    logits = lax.dot_general(x, rw, (((1,), (1,)), ((), ())),
                             preferred_element_type=jnp.float32)
    s = jax.nn.sigmoid(logits)
    T = s.shape[0]

    s3 = s.reshape(T, N_GROUP, GROUP_SIZE)
    io3 = lax.broadcasted_iota(jnp.int32, (T, N_GROUP, GROUP_SIZE), 2)
    m1 = jnp.max(s3, axis=2, keepdims=True)
    p1 = jnp.min(jnp.where(s3 == m1, io3, GROUP_SIZE), axis=2,
                 keepdims=True)
    s3b = jnp.where(io3 == p1, -jnp.inf, s3)
    m2 = jnp.max(s3b, axis=2, keepdims=True)
    gsc = (m1 + m2)[:, :, 0]

    iota_ng = lax.broadcasted_iota(jnp.int32, (T, N_GROUP), 1)
    sel = jnp.zeros((T, N_GROUP), jnp.float32)
    cur = gsc
    for _ in range(TOPK_GROUP):
        m = jnp.max(cur, axis=1, keepdims=True)
        p = jnp.min(jnp.where(cur == m, iota_ng, N_GROUP), axis=1,
                    keepdims=True)
        hit = iota_ng == p
        sel = jnp.where(hit, 1.0, sel)
        cur = jnp.where(hit, -jnp.inf, cur)

    masked = (s3 * sel[:, :, None]).reshape(T, E)
    iota_e = lax.broadcasted_iota(jnp.int32, (T, E), 1)
    m = jnp.max(masked, axis=1, keepdims=True)
    pos = jnp.min(jnp.where(masked == m, iota_e, E), axis=1, keepdims=True)
    wraw = jnp.sum(jnp.where(iota_e == pos, s, 0.0), axis=1, keepdims=True)
    wn = wraw / (wraw + 1e-20) * SCALE
    eid_ref[...] = pos
    w_ref[...] = wn


def _moe_body(ti, ei, fi, vi, x_ref, g_ref, u_ref, d_ref, eid_ref, sw_ref,
              out_ref):
    i = pl.program_id(0)
    e = ei[i]
    x = x_ref[...].astype(jnp.bfloat16)
    g = g_ref[0].astype(jnp.bfloat16)
    u = u_ref[0].astype(jnp.bfloat16)
    d = d_ref[0].astype(jnp.bfloat16)
    h1 = lax.dot_general(x, g, (((1,), (1,)), ((), ())),
                         preferred_element_type=jnp.float32)
    h2 = lax.dot_general(x, u, (((1,), (1,)), ((), ())),
                         preferred_element_type=jnp.float32)
    act = (h1 * jax.nn.sigmoid(h1) * h2).astype(jnp.bfloat16)
    part = lax.dot_general(act, d, (((1,), (1,)), ((), ())),
                           preferred_element_type=jnp.float32)
    match = jnp.logical_and(eid_ref[...] == e, vi[i] == 1)
    wcol = jnp.where(match, sw_ref[...], 0.0)

    @pl.when(fi[i] == 1)
    def _():
        out_ref[...] = jnp.zeros_like(out_ref)

    out_ref[...] += part * wcol


def _shared_body(x_ref, g_ref, u_ref, d_ref, m_ref, out_ref):
    x = x_ref[...].astype(jnp.bfloat16)
    h1 = lax.dot_general(x, g_ref[...].astype(jnp.bfloat16),
                         (((1,), (1,)), ((), ())),
                         preferred_element_type=jnp.float32)
    h2 = lax.dot_general(x, u_ref[...].astype(jnp.bfloat16),
                         (((1,), (1,)), ((), ())),
                         preferred_element_type=jnp.float32)
    act = (h1 * jax.nn.sigmoid(h1) * h2).astype(jnp.bfloat16)
    out = lax.dot_general(act, d_ref[...].astype(jnp.bfloat16),
                          (((1,), (1,)), ((), ())),
                          preferred_element_type=jnp.float32)
    out_ref[...] = out + m_ref[...]


def kernel(hidden_states, router_w, gate_w, up_w, down_w, sh_gate_w,
           sh_up_w, sh_down_w):
    orig_shape = hidden_states.shape
    hs = hidden_states.reshape(-1, H)
    T = hs.shape[0]

    eid2, w2 = pl.pallas_call(
        _router_body,
        grid=(T // RT,),
        in_specs=[
            pl.BlockSpec((RT, H), lambda i: (i, 0)),
            pl.BlockSpec((E, H), lambda i: (0, 0)),
        ],
        out_specs=[pl.BlockSpec((RT, 1), lambda i: (i, 0)),
                   pl.BlockSpec((RT, 1), lambda i: (i, 0))],
        out_shape=[jax.ShapeDtypeStruct((T, 1), jnp.int32),
                   jax.ShapeDtypeStruct((T, 1), jnp.float32)],
    )(hs, router_w)
    eid = eid2[:, 0]
    wtok = w2[:, 0]

    perm = jnp.argsort(eid).astype(jnp.int32)
    sorted_eid = eid[perm]
    sw = wtok[perm]
    hs_sorted = _sc_permute(hs, perm, scatter=False)

    NT = T // R
    e_lo = sorted_eid[::R]
    e_hi = sorted_eid[R - 1::R]
    cnt = e_hi - e_lo + 1
    cum = jnp.concatenate([jnp.zeros(1, cnt.dtype), jnp.cumsum(cnt)])
    total = cum[NT]
    W = NT + E
    i_arr = jnp.arange(W)
    r_i = jnp.sum((cum[None, :] <= i_arr[:, None]).astype(jnp.int32),
                  axis=1) - 1
    valid = i_arr < total
    r_c = jnp.minimum(r_i, NT - 1).astype(jnp.int32)
    e_raw = e_lo[r_c] + (i_arr - cum[r_c])
    e_i = jnp.where(valid, jnp.clip(e_raw, 0, E - 1),
                    sorted_eid[T - 1]).astype(jnp.int32)
    t_i = jnp.where(valid, r_c, NT - 1).astype(jnp.int32)
    first_i = jnp.logical_and(valid, i_arr == cum[r_c]).astype(jnp.int32)
    valid_i = valid.astype(jnp.int32)

    eid_2d = sorted_eid.reshape(T, 1)
    sw_2d = sw.reshape(T, 1)

    grid_spec = pltpu.PrefetchScalarGridSpec(
        num_scalar_prefetch=4,
        grid=(W,),
        in_specs=[
            pl.BlockSpec((R, H), lambda i, ti, ei, fi, vi: (ti[i], 0)),
            pl.BlockSpec((1, MOE_I, H),
                         lambda i, ti, ei, fi, vi: (ei[i], 0, 0)),
            pl.BlockSpec((1, MOE_I, H),
                         lambda i, ti, ei, fi, vi: (ei[i], 0, 0)),
            pl.BlockSpec((1, H, MOE_I),
                         lambda i, ti, ei, fi, vi: (ei[i], 0, 0)),
            pl.BlockSpec((R, 1), lambda i, ti, ei, fi, vi: (ti[i], 0)),
            pl.BlockSpec((R, 1), lambda i, ti, ei, fi, vi: (ti[i], 0)),
        ],
        out_specs=pl.BlockSpec((R, H), lambda i, ti, ei, fi, vi: (ti[i], 0)),
    )
    out_sorted = pl.pallas_call(
        _moe_body,
        grid_spec=grid_spec,
        out_shape=jax.ShapeDtypeStruct((T, H), jnp.float32),
        compiler_params=pltpu.CompilerParams(
            dimension_semantics=("arbitrary",)),
    )(t_i, e_i, first_i, valid_i, hs_sorted, gate_w, up_w, down_w,
      eid_2d, sw_2d)

    NT2 = T // RT
    I2 = sh_gate_w.shape[0]
    final_sorted = pl.pallas_call(
        _shared_body,
        grid=(NT2,),
        in_specs=[
            pl.BlockSpec((RT, H), lambda i: (i, 0)),
            pl.BlockSpec((I2, H), lambda i: (0, 0)),
            pl.BlockSpec((I2, H), lambda i: (0, 0)),
            pl.BlockSpec((H, I2), lambda i: (0, 0)),
            pl.BlockSpec((RT, H), lambda i: (i, 0)),
        ],
        out_specs=pl.BlockSpec((RT, H), lambda i: (i, 0)),
        out_shape=jax.ShapeDtypeStruct((T, H), jnp.float32),
    )(hs_sorted, sh_gate_w, sh_up_w, sh_down_w, out_sorted)

    final = _sc_permute(final_sorted, perm, scatter=True)

    return final.reshape(orig_shape)


# speedup vs baseline: 1.0228x; 1.0228x over previous
import functools

import jax
import jax.numpy as jnp
from jax import lax
from jax.experimental import pallas as pl
from jax.experimental.pallas import tpu as pltpu
from jax.experimental.pallas import tpu_sc as plsc

H = 1024
MOE_I = 512
E = 64
N_GROUP = 8
GROUP_SIZE = E // N_GROUP
TOPK_GROUP = 4
SCALE = 2.5

R = 256
RT = 256

_SC_NC = 2
_SC_NS = 16
_SC_NW = _SC_NC * _SC_NS


def _sc_worker_base(bpw):
    wid = lax.axis_index("s") * _SC_NC + lax.axis_index("c")
    return wid * bpw


def _sc_gather_body(bpw, table_hbm, idx_hbm, out_hbm, idx_v, rows_v, sem):
    base = _sc_worker_base(bpw)
    pltpu.sync_copy(idx_hbm.at[pl.ds(base, bpw)], idx_v)
    pltpu.async_copy(table_hbm.at[idx_v], rows_v, sem).wait()
    pltpu.sync_copy(rows_v, out_hbm.at[pl.ds(base, bpw)])


def _sc_scatter_body(bpw, rows_hbm, idx_hbm, out_hbm, idx_v, rows_v, sem):
    base = _sc_worker_base(bpw)
    pltpu.sync_copy(idx_hbm.at[pl.ds(base, bpw)], idx_v)
    pltpu.sync_copy(rows_hbm.at[pl.ds(base, bpw)], rows_v)
    pltpu.async_copy(rows_v, out_hbm.at[idx_v], sem).wait()


def _sc_permute(table, idx, scatter):
    n, d = table.shape
    bpw = n // _SC_NW
    body = functools.partial(_sc_scatter_body if scatter else _sc_gather_body,
                             bpw)
    mesh = plsc.VectorSubcoreMesh(core_axis_name="c", subcore_axis_name="s")
    return pl.kernel(
        body,
        mesh=mesh,
        out_type=jax.ShapeDtypeStruct((n, d), table.dtype),
        scratch_types=[
            pltpu.VMEM((bpw,), jnp.int32),
            pltpu.VMEM((bpw, d), table.dtype),
            pltpu.SemaphoreType.DMA,
        ],
    )(table, idx)


def _router_body(hs_ref, rw_ref, eid_ref, w_ref):
    x = hs_ref[...]
    rw = rw_ref[...]
    logits = lax.dot_general(x, rw, (((1,), (1,)), ((), ())),
                             preferred_element_type=jnp.float32)
    s = jax.nn.sigmoid(logits)
    T = s.shape[0]

    s3 = s.reshape(T, N_GROUP, GROUP_SIZE)
    io3 = lax.broadcasted_iota(jnp.int32, (T, N_GROUP, GROUP_SIZE), 2)
    m1 = jnp.max(s3, axis=2, keepdims=True)
    p1 = jnp.min(jnp.where(s3 == m1, io3, GROUP_SIZE), axis=2,
                 keepdims=True)
    s3b = jnp.where(io3 == p1, -jnp.inf, s3)
    m2 = jnp.max(s3b, axis=2, keepdims=True)
    gsc = (m1 + m2)[:, :, 0]

    iota_ng = lax.broadcasted_iota(jnp.int32, (T, N_GROUP), 1)
    sel = jnp.zeros((T, N_GROUP), jnp.float32)
    cur = gsc
    for _ in range(TOPK_GROUP):
        m = jnp.max(cur, axis=1, keepdims=True)
        p = jnp.min(jnp.where(cur == m, iota_ng, N_GROUP), axis=1,
                    keepdims=True)
        hit = iota_ng == p
        sel = jnp.where(hit, 1.0, sel)
        cur = jnp.where(hit, -jnp.inf, cur)

    masked = (s3 * sel[:, :, None]).reshape(T, E)
    iota_e = lax.broadcasted_iota(jnp.int32, (T, E), 1)
    m = jnp.max(masked, axis=1, keepdims=True)
    pos = jnp.min(jnp.where(masked == m, iota_e, E), axis=1, keepdims=True)
    wraw = jnp.sum(jnp.where(iota_e == pos, s, 0.0), axis=1, keepdims=True)
    wn = wraw / (wraw + 1e-20) * SCALE
    eid_ref[...] = pos
    w_ref[...] = wn


def _moe_body(ti, ei, fi, vi, x_ref, g_ref, u_ref, d_ref, eid_ref, sw_ref,
              out_ref):
    i = pl.program_id(0)
    e = ei[i]
    x = x_ref[...].astype(jnp.bfloat16)
    g = g_ref[0].astype(jnp.bfloat16)
    u = u_ref[0].astype(jnp.bfloat16)
    d = d_ref[0].astype(jnp.bfloat16)
    h1 = lax.dot_general(x, g, (((1,), (1,)), ((), ())),
                         preferred_element_type=jnp.float32)
    h2 = lax.dot_general(x, u, (((1,), (1,)), ((), ())),
                         preferred_element_type=jnp.float32)
    act = (h1 * jax.nn.sigmoid(h1) * h2).astype(jnp.bfloat16)
    part = lax.dot_general(act, d, (((1,), (1,)), ((), ())),
                           preferred_element_type=jnp.float32)
    match = jnp.logical_and(eid_ref[...] == e, vi[i] == 1)
    wcol = jnp.where(match, sw_ref[...], 0.0)

    @pl.when(fi[i] == 1)
    def _():
        out_ref[...] = jnp.zeros_like(out_ref)

    out_ref[...] += part * wcol


def _shared_body(x_ref, g_ref, u_ref, d_ref, m_ref, out_ref):
    x = x_ref[...].astype(jnp.bfloat16)
    h1 = lax.dot_general(x, g_ref[...].astype(jnp.bfloat16),
                         (((1,), (1,)), ((), ())),
                         preferred_element_type=jnp.float32)
    h2 = lax.dot_general(x, u_ref[...].astype(jnp.bfloat16),
                         (((1,), (1,)), ((), ())),
                         preferred_element_type=jnp.float32)
    act = (h1 * jax.nn.sigmoid(h1) * h2).astype(jnp.bfloat16)
    out = lax.dot_general(act, d_ref[...].astype(jnp.bfloat16),
                          (((1,), (1,)), ((), ())),
                          preferred_element_type=jnp.float32)
    out_ref[...] = out + m_ref[...]


def kernel(hidden_states, router_w, gate_w, up_w, down_w, sh_gate_w,
           sh_up_w, sh_down_w):
    orig_shape = hidden_states.shape
    hs = hidden_states.reshape(-1, H)
    T = hs.shape[0]

    eid2, w2 = pl.pallas_call(
        _router_body,
        out_shape=[jax.ShapeDtypeStruct((T, 1), jnp.int32),
                   jax.ShapeDtypeStruct((T, 1), jnp.float32)],
    )(hs, router_w)
    eid = eid2[:, 0]
    wtok = w2[:, 0]

    perm = jnp.argsort(eid).astype(jnp.int32)
    sorted_eid = eid[perm]
    sw = wtok[perm]
    hs_sorted = _sc_permute(hs, perm, scatter=False)

    NT = T // R
    e_lo = sorted_eid[::R]
    e_hi = sorted_eid[R - 1::R]
    cnt = e_hi - e_lo + 1
    cum = jnp.concatenate([jnp.zeros(1, cnt.dtype), jnp.cumsum(cnt)])
    total = cum[NT]
    W = NT + E
    i_arr = jnp.arange(W)
    r_i = jnp.sum((cum[None, :] <= i_arr[:, None]).astype(jnp.int32),
                  axis=1) - 1
    valid = i_arr < total
    r_c = jnp.minimum(r_i, NT - 1).astype(jnp.int32)
    e_raw = e_lo[r_c] + (i_arr - cum[r_c])
    e_i = jnp.where(valid, jnp.clip(e_raw, 0, E - 1),
                    sorted_eid[T - 1]).astype(jnp.int32)
    t_i = jnp.where(valid, r_c, NT - 1).astype(jnp.int32)
    first_i = jnp.logical_and(valid, i_arr == cum[r_c]).astype(jnp.int32)
    valid_i = valid.astype(jnp.int32)

    eid_2d = sorted_eid.reshape(T, 1)
    sw_2d = sw.reshape(T, 1)

    grid_spec = pltpu.PrefetchScalarGridSpec(
        num_scalar_prefetch=4,
        grid=(W,),
        in_specs=[
            pl.BlockSpec((R, H), lambda i, ti, ei, fi, vi: (ti[i], 0)),
            pl.BlockSpec((1, MOE_I, H),
                         lambda i, ti, ei, fi, vi: (ei[i], 0, 0)),
            pl.BlockSpec((1, MOE_I, H),
                         lambda i, ti, ei, fi, vi: (ei[i], 0, 0)),
            pl.BlockSpec((1, H, MOE_I),
                         lambda i, ti, ei, fi, vi: (ei[i], 0, 0)),
            pl.BlockSpec((R, 1), lambda i, ti, ei, fi, vi: (ti[i], 0)),
            pl.BlockSpec((R, 1), lambda i, ti, ei, fi, vi: (ti[i], 0)),
        ],
        out_specs=pl.BlockSpec((R, H), lambda i, ti, ei, fi, vi: (ti[i], 0)),
    )
    out_sorted = pl.pallas_call(
        _moe_body,
        grid_spec=grid_spec,
        out_shape=jax.ShapeDtypeStruct((T, H), jnp.float32),
        compiler_params=pltpu.CompilerParams(
            dimension_semantics=("arbitrary",)),
    )(t_i, e_i, first_i, valid_i, hs_sorted, gate_w, up_w, down_w,
      eid_2d, sw_2d)

    NT2 = T // RT
    I2 = sh_gate_w.shape[0]
    final_sorted = pl.pallas_call(
        _shared_body,
        grid=(NT2,),
        in_specs=[
            pl.BlockSpec((RT, H), lambda i: (i, 0)),
            pl.BlockSpec((I2, H), lambda i: (0, 0)),
            pl.BlockSpec((I2, H), lambda i: (0, 0)),
            pl.BlockSpec((H, I2), lambda i: (0, 0)),
            pl.BlockSpec((RT, H), lambda i: (i, 0)),
        ],
        out_specs=pl.BlockSpec((RT, H), lambda i: (i, 0)),
        out_shape=jax.ShapeDtypeStruct((T, H), jnp.float32),
    )(hs_sorted, sh_gate_w, sh_up_w, sh_down_w, out_sorted)

    final = _sc_permute(final_sorted, perm, scatter=True)

    return final.reshape(orig_shape)
